# baseline (speedup 1.0000x reference)
import functools

import jax
import jax.numpy as jnp
from jax import lax
from jax.experimental import pallas as pl
from jax.experimental.pallas import tpu as pltpu
from jax.experimental.pallas import tpu_sc as plsc

B, S = 4, 8192
MIN_SPECIAL = 50256
MASK_TOKEN = 50257

NC, NS, L = 1, 16, 16
NW = NC * NS
CHUNKS_PER_ROW = NW // B
CHUNK = S // CHUNKS_PER_ROW
HALF = CHUNK // 2
NVEC_H = HALF // L

_mesh = plsc.VectorSubcoreMesh(core_axis_name="c", subcore_axis_name="s", num_cores=NC)


@functools.partial(
    pl.kernel,
    mesh=_mesh,
    out_type=jax.ShapeDtypeStruct((B, S), jnp.int32),
    scratch_types=[
        pltpu.VMEM((CHUNK,), jnp.int32),
        pltpu.VMEM((CHUNK,), jnp.int32),
        pltpu.VMEM((CHUNK,), jnp.int32),
        pltpu.SemaphoreType.DMA,
        pltpu.SemaphoreType.DMA,
        pltpu.SemaphoreType.DMA,
    ],
)
def _mlm_sc(w_hbm, rt_hbm, out_hbm,
            w_v, rt_v, out_v, sem0, sem1, sem_out):
    wid = lax.axis_index("s") * NC + lax.axis_index("c")
    row = wid // CHUNKS_PER_ROW
    col = (wid % CHUNKS_PER_ROW) * CHUNK

    zero = jnp.zeros((L,), jnp.int32)
    low16 = jnp.full((L,), 0xFFFF, jnp.int32)
    mask_tok = jnp.full((L,), MASK_TOKEN, jnp.int32)

    copies = []
    for h, sem in ((0, sem0), (1, sem1)):
        sl_h = pl.ds(col + h * HALF, HALF)
        sl_v = pl.ds(h * HALF, HALF)
        copies.append((
            pltpu.async_copy(w_hbm.at[row, sl_h], w_v.at[sl_v], sem),
            pltpu.async_copy(rt_hbm.at[row, sl_h], rt_v.at[sl_v], sem),
        ))

    out_copies = []
    for h in (0, 1):
        for cp in copies[h]:
            cp.wait()
        base = h * HALF
        for j in range(NVEC_H):
            sl = pl.ds(base + j * L, L)
            w = w_v[sl]
            x = w & low16
            masked = (w < zero) & (x < MIN_SPECIAL)
            y = jnp.where(masked, mask_tok, x)
            y = jnp.where((w << 1) < zero, rt_v[sl], y)
            out_v[sl] = y
        out_copies.append(pltpu.async_copy(
            out_v.at[pl.ds(base, HALF)],
            out_hbm.at[row, pl.ds(col + base, HALF)], sem_out))

    for cp in out_copies:
        cp.wait()


def kernel(inputs, input_masks_2, input_masks_3, random_tokens, loss_weight):
    w = inputs | (input_masks_2.astype(jnp.int32) << 31) | (
        input_masks_3.astype(jnp.int32) << 30)
    out = _mlm_sc(w, random_tokens)
    return out, loss_weight


# speedup vs baseline: 1.0218x; 1.0218x over previous
import functools

import jax
import jax.numpy as jnp
from jax import lax
from jax.experimental import pallas as pl
from jax.experimental.pallas import tpu as pltpu
from jax.experimental.pallas import tpu_sc as plsc

B, S = 4, 8192
MIN_SPECIAL = 50256
MASK_TOKEN = 50257

NC, NS, L = 1, 16, 16
NW = NC * NS
CHUNKS_PER_ROW = NW // B
CHUNK = S // CHUNKS_PER_ROW
HALF = CHUNK // 2
NVEC_H = HALF // L

_mesh = plsc.VectorSubcoreMesh(core_axis_name="c", subcore_axis_name="s", num_cores=NC)


@functools.partial(
    pl.kernel,
    mesh=_mesh,
    out_type=jax.ShapeDtypeStruct((B, S), jnp.int32),
    scratch_types=[
        pltpu.VMEM((CHUNK,), jnp.int32),
        pltpu.VMEM((CHUNK,), jnp.int32),
        pltpu.SemaphoreType.DMA,
        pltpu.SemaphoreType.DMA,
        pltpu.SemaphoreType.DMA,
    ],
)
def _mlm_sc(w_hbm, out_hbm,
            w_v, out_v, sem0, sem1, sem_out):
    wid = lax.axis_index("s") * NC + lax.axis_index("c")
    row = wid // CHUNKS_PER_ROW
    col = (wid % CHUNKS_PER_ROW) * CHUNK

    zero = jnp.zeros((L,), jnp.int32)
    low16 = jnp.full((L,), 0xFFFF, jnp.int32)
    mask_tok = jnp.full((L,), MASK_TOKEN, jnp.int32)

    copies = []
    for h, sem in ((0, sem0), (1, sem1)):
        sl_h = pl.ds(col + h * HALF, HALF)
        sl_v = pl.ds(h * HALF, HALF)
        copies.append((
            pltpu.async_copy(w_hbm.at[row, sl_h], w_v.at[sl_v], sem),
        ))

    out_copies = []
    for h in (0, 1):
        for cp in copies[h]:
            cp.wait()
        base = h * HALF
        for j in range(NVEC_H):
            sl = pl.ds(base + j * L, L)
            w = w_v[sl]
            x = w & low16
            masked = (w < zero) & (x < MIN_SPECIAL)
            y = jnp.where(masked, mask_tok, x)
            y = jnp.where((w << 1) < zero, x, y)
            out_v[sl] = y
        out_copies.append(pltpu.async_copy(
            out_v.at[pl.ds(base, HALF)],
            out_hbm.at[row, pl.ds(col + base, HALF)], sem_out))

    for cp in out_copies:
        cp.wait()


def kernel(inputs, input_masks_2, input_masks_3, random_tokens, loss_weight):
    w = jnp.where(
        input_masks_3,
        random_tokens | (1 << 30),
        inputs | (input_masks_2.astype(jnp.int32) << 31),
    )
    out = _mlm_sc(w)
    return out, loss_weight
